# baseline (speedup 1.0000x reference)
import functools

import jax
import jax.numpy as jnp
from jax import lax
from jax.experimental import pallas as pl
from jax.experimental.pallas import tpu as pltpu
from jax.experimental.pallas import tpu_sc as plsc

V = 1000
N = 64 * 256
NC, NS = 2, 16
NW = NC * NS
ROWS_PER_W = N // NW
CH = 16
NCH = ROWS_PER_W // CH
TV = V // 8


def _logz_body(e_ref, o_ref):
    x = e_ref[...]
    m = jnp.max(x, axis=1)
    s = jnp.sum(jnp.exp(x - m[:, None]), axis=1)
    o_ref[...] = m + jnp.log(s)


def _loss_body(p_ref, o_ref):
    o_ref[0, 0] = jnp.sum(p_ref[...]) * (1.0 / N)


def _sc_body(embed, idx, tgt, logz, out, partials,
             idx_v, tgt_v, logz_v, rows0, rows1, stg0, stg1, acc_v,
             sem_g0, sem_g1, sem_o0, sem_o1):
    wid = lax.axis_index("s") * NC + lax.axis_index("c")
    base = wid * ROWS_PER_W
    pltpu.sync_copy(idx.at[pl.ds(base, ROWS_PER_W)], idx_v)
    pltpu.sync_copy(tgt.at[pl.ds(base, ROWS_PER_W)], tgt_v)
    pltpu.sync_copy(logz, logz_v)

    iota = jnp.arange(16, dtype=jnp.int32)
    rows = [rows0, rows1]
    sems = [sem_g0, sem_g1]
    stgs = [stg0, stg1]
    sems_o = [sem_o0, sem_o1]

    acc = jnp.zeros((16,), jnp.float32)
    for g in range(ROWS_PER_W // 16):
        acc = acc + plsc.load_gather(logz_v, [idx_v[pl.ds(g * 16, 16)]])

    def issue_gather(u, p):
        return pltpu.async_copy(
            embed.at[idx_v.at[pl.ds(u * CH, CH)]], rows[p], sems[p])

    def chunk_body(u, p, acc):
        pltpu.make_async_copy(
            embed.at[idx_v.at[pl.ds(u * CH, CH)]], rows[p], sems[p]).wait()

        stg = stgs[p]
        tc = wid * (ROWS_PER_W // 128) + u // 8
        l0 = (u - (u // 8) * 8) * CH

        @pl.when(u >= 2)
        def _():
            pltpu.make_async_copy(
                stg, out.at[:, tc, :, pl.ds(l0, CH)], sems_o[p]).wait()

        @plsc.parallel_loop(0, TV, step=1, unroll=6)
        def tile_body(t):
            for r in range(8):
                v = plsc.load_gather(
                    rows[p], [iota, jnp.full((16,), t * 8 + r, jnp.int32)])
                stg[t, r, :] = v

        tgt16 = tgt_v[pl.ds(u * CH, CH)]
        acc = acc - plsc.load_gather(rows[p], [iota, tgt16])

        pltpu.async_copy(stg, out.at[:, tc, :, pl.ds(l0, CH)], sems_o[p])
        return acc

    issue_gather(jnp.int32(0), 0)

    def pair_body(k, acc):
        u0 = 2 * k
        issue_gather(u0 + 1, 1)
        acc = chunk_body(u0, 0, acc)

        @pl.when(u0 + 2 < NCH)
        def _():
            issue_gather(u0 + 2, 0)

        acc = chunk_body(u0 + 1, 1, acc)
        return acc

    acc = lax.fori_loop(0, NCH // 2, pair_body, acc, unroll=False)

    pltpu.make_async_copy(stg0, out.at[:, 0, :, pl.ds(0, CH)], sem_o0).wait()
    pltpu.make_async_copy(stg1, out.at[:, 0, :, pl.ds(0, CH)], sem_o1).wait()

    acc_v[...] = acc
    pltpu.sync_copy(acc_v, partials.at[wid])


_sc_gather = functools.partial(
    pl.kernel,
    mesh=plsc.VectorSubcoreMesh(core_axis_name="c", subcore_axis_name="s"),
    compiler_params=pltpu.CompilerParams(
        use_tc_tiling_on_sc=False, needs_layout_passes=False),
    out_type=[
        jax.ShapeDtypeStruct((TV, N // 128, 8, 128), jnp.float32),
        jax.ShapeDtypeStruct((NW, 16), jnp.float32),
    ],
    scratch_types=[
        pltpu.VMEM((ROWS_PER_W,), jnp.int32),
        pltpu.VMEM((ROWS_PER_W,), jnp.int32),
        pltpu.VMEM((V,), jnp.float32),
        pltpu.VMEM((CH, V), jnp.float32),
        pltpu.VMEM((CH, V), jnp.float32),
        pltpu.VMEM((TV, 8, CH), jnp.float32),
        pltpu.VMEM((TV, 8, CH), jnp.float32),
        pltpu.VMEM((16,), jnp.float32),
        pltpu.SemaphoreType.DMA,
        pltpu.SemaphoreType.DMA,
        pltpu.SemaphoreType.DMA,
        pltpu.SemaphoreType.DMA,
    ],
)(_sc_body)


def kernel(input, target, embed):
    idx = input.reshape(-1).astype(jnp.int32)
    tgt = target.reshape(-1).astype(jnp.int32)
    logz = pl.pallas_call(
        _logz_body,
        out_shape=jax.ShapeDtypeStruct((V,), jnp.float32),
    )(embed)
    tiles, partials = _sc_gather(embed, idx, tgt, logz)
    logit2 = tiles.transpose(1, 3, 0, 2).reshape(N, V)
    loss2d = pl.pallas_call(
        _loss_body,
        out_shape=jax.ShapeDtypeStruct((1, 1), jnp.float32),
        out_specs=pl.BlockSpec(memory_space=pltpu.SMEM),
    )(partials)
    return (logit2, loss2d[0, 0])


# speedup vs baseline: 1.0402x; 1.0402x over previous
import functools

import jax
import jax.numpy as jnp
from jax import lax
from jax.experimental import pallas as pl
from jax.experimental.pallas import tpu as pltpu
from jax.experimental.pallas import tpu_sc as plsc

V = 1000
N = 64 * 256
NC, NS = 2, 16
NW = NC * NS
ROWS_PER_W = N // NW
CH = 16
NCH = ROWS_PER_W // CH
TV = V // 8


def _logz_body(e_ref, o_ref):
    x = e_ref[...]
    m = jnp.max(x, axis=1)
    s = jnp.sum(jnp.exp(x - m[:, None]), axis=1)
    o_ref[...] = m + jnp.log(s)


def _loss_body(p_ref, o_ref):
    o_ref[0, 0] = jnp.sum(p_ref[...]) * (1.0 / N)


def _sc_body(embed, idx, tgt, logz, out, partials,
             idx_v, tgt_v, logz_v, rows0, rows1, stg0, stg1, acc_v,
             sem_g0, sem_g1, sem_o0, sem_o1):
    wid = lax.axis_index("s") * NC + lax.axis_index("c")
    base = wid * ROWS_PER_W
    pltpu.sync_copy(idx.at[pl.ds(base, ROWS_PER_W)], idx_v)
    pltpu.sync_copy(tgt.at[pl.ds(base, ROWS_PER_W)], tgt_v)
    pltpu.sync_copy(logz, logz_v)

    iota = jnp.arange(16, dtype=jnp.int32)
    rows = [rows0, rows1]
    sems = [sem_g0, sem_g1]
    stgs = [stg0, stg1]
    sems_o = [sem_o0, sem_o1]

    acc = jnp.zeros((16,), jnp.float32)
    for g in range(ROWS_PER_W // 16):
        acc = acc + plsc.load_gather(logz_v, [idx_v[pl.ds(g * 16, 16)]])

    def issue_gather(u, p):
        return pltpu.async_copy(
            embed.at[idx_v.at[pl.ds(u * CH, CH)]], rows[p], sems[p])

    def chunk_body(u, p, acc):
        pltpu.make_async_copy(
            embed.at[idx_v.at[pl.ds(u * CH, CH)]], rows[p], sems[p]).wait()

        stg = stgs[p]
        tc = wid * (ROWS_PER_W // 128) + u // 8
        l0 = (u - (u // 8) * 8) * CH

        @pl.when(u >= 2)
        def _():
            pltpu.make_async_copy(
                stg, out.at[:, tc, :, pl.ds(l0, CH)], sems_o[p]).wait()

        @plsc.parallel_loop(0, TV, step=1, unroll=3)
        def tile_body(t):
            for r in range(8):
                v = plsc.load_gather(
                    rows[p], [iota, jnp.full((16,), t * 8 + r, jnp.int32)])
                stg[t, r, :] = v

        tgt16 = tgt_v[pl.ds(u * CH, CH)]
        acc = acc - plsc.load_gather(rows[p], [iota, tgt16])

        pltpu.async_copy(stg, out.at[:, tc, :, pl.ds(l0, CH)], sems_o[p])
        return acc

    issue_gather(jnp.int32(0), 0)

    def pair_body(k, acc):
        u0 = 2 * k
        issue_gather(u0 + 1, 1)
        acc = chunk_body(u0, 0, acc)

        @pl.when(u0 + 2 < NCH)
        def _():
            issue_gather(u0 + 2, 0)

        acc = chunk_body(u0 + 1, 1, acc)
        return acc

    acc = lax.fori_loop(0, NCH // 2, pair_body, acc, unroll=False)

    pltpu.make_async_copy(stg0, out.at[:, 0, :, pl.ds(0, CH)], sem_o0).wait()
    pltpu.make_async_copy(stg1, out.at[:, 0, :, pl.ds(0, CH)], sem_o1).wait()

    acc_v[...] = acc
    pltpu.sync_copy(acc_v, partials.at[wid])


_sc_gather = functools.partial(
    pl.kernel,
    mesh=plsc.VectorSubcoreMesh(core_axis_name="c", subcore_axis_name="s"),
    compiler_params=pltpu.CompilerParams(
        use_tc_tiling_on_sc=False, needs_layout_passes=False),
    out_type=[
        jax.ShapeDtypeStruct((TV, N // 128, 8, 128), jnp.float32),
        jax.ShapeDtypeStruct((NW, 16), jnp.float32),
    ],
    scratch_types=[
        pltpu.VMEM((ROWS_PER_W,), jnp.int32),
        pltpu.VMEM((ROWS_PER_W,), jnp.int32),
        pltpu.VMEM((V,), jnp.float32),
        pltpu.VMEM((CH, V), jnp.float32),
        pltpu.VMEM((CH, V), jnp.float32),
        pltpu.VMEM((TV, 8, CH), jnp.float32),
        pltpu.VMEM((TV, 8, CH), jnp.float32),
        pltpu.VMEM((16,), jnp.float32),
        pltpu.SemaphoreType.DMA,
        pltpu.SemaphoreType.DMA,
        pltpu.SemaphoreType.DMA,
        pltpu.SemaphoreType.DMA,
    ],
)(_sc_body)


def kernel(input, target, embed):
    idx = input.reshape(-1).astype(jnp.int32)
    tgt = target.reshape(-1).astype(jnp.int32)
    logz = pl.pallas_call(
        _logz_body,
        out_shape=jax.ShapeDtypeStruct((V,), jnp.float32),
    )(embed)
    tiles, partials = _sc_gather(embed, idx, tgt, logz)
    logit2 = tiles.transpose(1, 3, 0, 2).reshape(N, V)
    loss2d = pl.pallas_call(
        _loss_body,
        out_shape=jax.ShapeDtypeStruct((1, 1), jnp.float32),
        out_specs=pl.BlockSpec(memory_space=pltpu.SMEM),
    )(partials)
    return (logit2, loss2d[0, 0])


# speedup vs baseline: 1.0449x; 1.0045x over previous
import functools

import jax
import jax.numpy as jnp
from jax import lax
from jax.experimental import pallas as pl
from jax.experimental.pallas import tpu as pltpu
from jax.experimental.pallas import tpu_sc as plsc

V = 1000
N = 64 * 256
NC, NS = 2, 16
NW = NC * NS
ROWS_PER_W = N // NW
CH = 16
NCH = ROWS_PER_W // CH
TV = V // 8


def _logz_body(e_ref, o_ref):
    x = e_ref[...]
    m = jnp.max(x, axis=1)
    s = jnp.sum(jnp.exp(x - m[:, None]), axis=1)
    o_ref[...] = m + jnp.log(s)


def _loss_body(p_ref, o_ref):
    o_ref[0, 0] = jnp.sum(p_ref[...]) * (1.0 / N)


def _sc_body(embed, idx, tgt, logz, out, partials,
             idx_v, tgt_v, logz_v, rows0, rows1, stg0, stg1, acc_v,
             sem_g0, sem_g1, sem_o0, sem_o1):
    wid = lax.axis_index("s") * NC + lax.axis_index("c")
    base = wid * ROWS_PER_W
    pltpu.sync_copy(idx.at[pl.ds(base, ROWS_PER_W)], idx_v)
    pltpu.sync_copy(tgt.at[pl.ds(base, ROWS_PER_W)], tgt_v)
    pltpu.sync_copy(logz, logz_v)

    iota = jnp.arange(16, dtype=jnp.int32)
    rows = [rows0, rows1]
    sems = [sem_g0, sem_g1]
    stgs = [stg0, stg1]
    sems_o = [sem_o0, sem_o1]

    acc = jnp.zeros((16,), jnp.float32)
    for g in range(ROWS_PER_W // 16):
        acc = acc + plsc.load_gather(logz_v, [idx_v[pl.ds(g * 16, 16)]])

    def issue_gather(u, p):
        return pltpu.async_copy(
            embed.at[idx_v.at[pl.ds(u * CH, CH)]], rows[p], sems[p])

    def chunk_body(u, p, acc):
        pltpu.make_async_copy(
            embed.at[idx_v.at[pl.ds(u * CH, CH)]], rows[p], sems[p]).wait()

        stg = stgs[p]
        tc = wid * (ROWS_PER_W // 128) + u // 8
        l0 = (u - (u // 8) * 8) * CH

        @pl.when(u >= 2)
        def _():
            pltpu.make_async_copy(
                stg, out.at[:, tc, :, pl.ds(l0, CH)], sems_o[p]).wait()

        @plsc.parallel_loop(0, TV, step=1, unroll=4)
        def tile_body(t):
            for r in range(8):
                v = plsc.load_gather(
                    rows[p], [iota, jnp.full((16,), t * 8 + r, jnp.int32)])
                stg[t, r, :] = v

        tgt16 = tgt_v[pl.ds(u * CH, CH)]
        acc = acc - plsc.load_gather(rows[p], [iota, tgt16])

        pltpu.async_copy(stg, out.at[:, tc, :, pl.ds(l0, CH)], sems_o[p])
        return acc

    issue_gather(jnp.int32(0), 0)

    def pair_body(k, acc):
        u0 = 2 * k
        issue_gather(u0 + 1, 1)
        acc = chunk_body(u0, 0, acc)

        @pl.when(u0 + 2 < NCH)
        def _():
            issue_gather(u0 + 2, 0)

        acc = chunk_body(u0 + 1, 1, acc)
        return acc

    acc = lax.fori_loop(0, NCH // 2, pair_body, acc, unroll=False)

    pltpu.make_async_copy(stg0, out.at[:, 0, :, pl.ds(0, CH)], sem_o0).wait()
    pltpu.make_async_copy(stg1, out.at[:, 0, :, pl.ds(0, CH)], sem_o1).wait()

    acc_v[...] = acc
    pltpu.sync_copy(acc_v, partials.at[wid])


_sc_gather = functools.partial(
    pl.kernel,
    mesh=plsc.VectorSubcoreMesh(core_axis_name="c", subcore_axis_name="s"),
    compiler_params=pltpu.CompilerParams(
        use_tc_tiling_on_sc=False, needs_layout_passes=False),
    out_type=[
        jax.ShapeDtypeStruct((TV, N // 128, 8, 128), jnp.float32),
        jax.ShapeDtypeStruct((NW, 16), jnp.float32),
    ],
    scratch_types=[
        pltpu.VMEM((ROWS_PER_W,), jnp.int32),
        pltpu.VMEM((ROWS_PER_W,), jnp.int32),
        pltpu.VMEM((V,), jnp.float32),
        pltpu.VMEM((CH, V), jnp.float32),
        pltpu.VMEM((CH, V), jnp.float32),
        pltpu.VMEM((TV, 8, CH), jnp.float32),
        pltpu.VMEM((TV, 8, CH), jnp.float32),
        pltpu.VMEM((16,), jnp.float32),
        pltpu.SemaphoreType.DMA,
        pltpu.SemaphoreType.DMA,
        pltpu.SemaphoreType.DMA,
        pltpu.SemaphoreType.DMA,
    ],
)(_sc_body)


def kernel(input, target, embed):
    idx = input.reshape(-1).astype(jnp.int32)
    tgt = target.reshape(-1).astype(jnp.int32)
    logz = pl.pallas_call(
        _logz_body,
        out_shape=jax.ShapeDtypeStruct((V,), jnp.float32),
    )(embed)
    tiles, partials = _sc_gather(embed, idx, tgt, logz)
    logit2 = tiles.transpose(1, 3, 0, 2).reshape(N, V)
    loss2d = pl.pallas_call(
        _loss_body,
        out_shape=jax.ShapeDtypeStruct((1, 1), jnp.float32),
        out_specs=pl.BlockSpec(memory_space=pltpu.SMEM),
    )(partials)
    return (logit2, loss2d[0, 0])


# speedup vs baseline: 1.0614x; 1.0157x over previous
import functools

import jax
import jax.numpy as jnp
from jax import lax
from jax.experimental import pallas as pl
from jax.experimental.pallas import tpu as pltpu
from jax.experimental.pallas import tpu_sc as plsc

V = 1000
N = 64 * 256
NC, NS = 2, 16
NW = NC * NS
ROWS_PER_W = N // NW
CH = 16
NCH = ROWS_PER_W // CH
TV = V // 8


def _logz_body(e_ref, o_ref):
    x = e_ref[...]
    m = jnp.max(x, axis=1)
    s = jnp.sum(jnp.exp(x - m[:, None]), axis=1)
    o_ref[...] = m + jnp.log(s)


def _loss_body(p_ref, o_ref):
    o_ref[0, 0] = jnp.sum(p_ref[...]) * (1.0 / N)


def _sc_body(embed, idx, tgt, logz, out, partials,
             idx_v, tgt_v, logz_v, rows0, rows1, stg0, stg1, acc_v,
             sem_g0, sem_g1, sem_o0, sem_o1):
    wid = lax.axis_index("s") * NC + lax.axis_index("c")
    base = wid * ROWS_PER_W
    pltpu.sync_copy(idx.at[pl.ds(base, ROWS_PER_W)], idx_v)
    pltpu.sync_copy(tgt.at[pl.ds(base, ROWS_PER_W)], tgt_v)
    pltpu.sync_copy(logz, logz_v)

    iota = jnp.arange(16, dtype=jnp.int32)
    rows = [rows0, rows1]
    sems = [sem_g0, sem_g1]
    stgs = [stg0, stg1]
    sems_o = [sem_o0, sem_o1]

    acc = jnp.zeros((16,), jnp.float32)
    for g in range(ROWS_PER_W // 16):
        acc = acc + plsc.load_gather(logz_v, [idx_v[pl.ds(g * 16, 16)]])

    def issue_gather(u, p):
        return pltpu.async_copy(
            embed.at[idx_v.at[pl.ds(u * CH, CH)]], rows[p], sems[p])

    def chunk_body(u, p, acc):
        pltpu.make_async_copy(
            embed.at[idx_v.at[pl.ds(u * CH, CH)]], rows[p], sems[p]).wait()

        stg = stgs[p]
        tc = wid * (ROWS_PER_W // 128) + u // 8
        l0 = (u - (u // 8) * 8) * CH

        @pl.when(u >= 2)
        def _():
            pltpu.make_async_copy(
                stg, out.at[:, tc, :, pl.ds(l0, CH)], sems_o[p]).wait()

        @plsc.parallel_loop(0, TV, step=1, unroll=2)
        def tile_body(t):
            for r in range(8):
                v = plsc.load_gather(
                    rows[p], [iota, jnp.full((16,), t * 8 + r, jnp.int32)])
                stg[t, r, :] = v

        tgt16 = tgt_v[pl.ds(u * CH, CH)]
        acc = acc - plsc.load_gather(rows[p], [iota, tgt16])

        pltpu.async_copy(stg, out.at[:, tc, :, pl.ds(l0, CH)], sems_o[p])
        return acc

    issue_gather(jnp.int32(0), 0)

    def pair_body(k, acc):
        u0 = 2 * k
        issue_gather(u0 + 1, 1)
        acc = chunk_body(u0, 0, acc)

        @pl.when(u0 + 2 < NCH)
        def _():
            issue_gather(u0 + 2, 0)

        acc = chunk_body(u0 + 1, 1, acc)
        return acc

    acc = lax.fori_loop(0, NCH // 2, pair_body, acc, unroll=False)

    pltpu.make_async_copy(stg0, out.at[:, 0, :, pl.ds(0, CH)], sem_o0).wait()
    pltpu.make_async_copy(stg1, out.at[:, 0, :, pl.ds(0, CH)], sem_o1).wait()

    acc_v[...] = acc
    pltpu.sync_copy(acc_v, partials.at[wid])


_sc_gather = functools.partial(
    pl.kernel,
    mesh=plsc.VectorSubcoreMesh(core_axis_name="c", subcore_axis_name="s"),
    compiler_params=pltpu.CompilerParams(
        use_tc_tiling_on_sc=False, needs_layout_passes=False),
    out_type=[
        jax.ShapeDtypeStruct((TV, N // 128, 8, 128), jnp.float32),
        jax.ShapeDtypeStruct((NW, 16), jnp.float32),
    ],
    scratch_types=[
        pltpu.VMEM((ROWS_PER_W,), jnp.int32),
        pltpu.VMEM((ROWS_PER_W,), jnp.int32),
        pltpu.VMEM((V,), jnp.float32),
        pltpu.VMEM((CH, V), jnp.float32),
        pltpu.VMEM((CH, V), jnp.float32),
        pltpu.VMEM((TV, 8, CH), jnp.float32),
        pltpu.VMEM((TV, 8, CH), jnp.float32),
        pltpu.VMEM((16,), jnp.float32),
        pltpu.SemaphoreType.DMA,
        pltpu.SemaphoreType.DMA,
        pltpu.SemaphoreType.DMA,
        pltpu.SemaphoreType.DMA,
    ],
)(_sc_body)


def kernel(input, target, embed):
    idx = input.reshape(-1).astype(jnp.int32)
    tgt = target.reshape(-1).astype(jnp.int32)
    logz = pl.pallas_call(
        _logz_body,
        out_shape=jax.ShapeDtypeStruct((V,), jnp.float32),
    )(embed)
    tiles, partials = _sc_gather(embed, idx, tgt, logz)
    logit2 = tiles.transpose(1, 3, 0, 2).reshape(N, V)
    loss2d = pl.pallas_call(
        _loss_body,
        out_shape=jax.ShapeDtypeStruct((1, 1), jnp.float32),
        out_specs=pl.BlockSpec(memory_space=pltpu.SMEM),
    )(partials)
    return (logit2, loss2d[0, 0])
